# baseline (speedup 1.0000x reference)
import functools

import jax
import jax.numpy as jnp
from jax import lax
from jax.experimental import pallas as pl
from jax.experimental.pallas import tpu as pltpu
from jax.experimental.pallas import tpu_sc as plsc

import numpy as np

_FACTOR = 5
_NEG = np.float32(-3.0e38)

_SAMP = np.array([1220, 18, 1207, 1217, 653, 1387, 385, 295, 6, 1282, 552,
                  2034, 1433, 475, 1996, 1810, 1611, 898, 835, 519, 1590,
                  651, 268, 1731, 1132, 1553, 1008, 539, 284, 1335, 261,
                  676, 1493, 46, 1075, 20, 814, 1970, 1873, 2029],
                 dtype=np.int32)


def _stage1_body(qt_ref, ks_ref, m_ref, *, seq_len, n_heads):
    ksb = ks_ref[0]
    inv = 1.0 / seq_len
    for h in range(n_heads):
        qt_h = qt_ref[0, h]
        s = lax.dot_general(ksb, qt_h, (((1,), (0,)), ((), ())),
                            preferred_element_type=jnp.float32)
        m_ref[0, 0, pl.ds(h * qt_h.shape[1], qt_h.shape[1])] = (
            jnp.max(s, axis=0) - jnp.sum(s, axis=0) * inv)


def _stage1(qt, ks):
    B, HV, D, L = qt.shape
    P = B * HV
    U = ks.shape[1]
    LBLK = L // HV
    return pl.pallas_call(
        functools.partial(_stage1_body, seq_len=L, n_heads=HV),
        grid=(P,),
        in_specs=[
            pl.BlockSpec((1, HV, D, LBLK), lambda i: (i // 16, 0, 0, i % 16)),
            pl.BlockSpec((1, U, D), lambda i: (i, 0, 0)),
        ],
        out_specs=pl.BlockSpec((1, 1, L), lambda i: (i, 0, 0)),
        out_shape=jax.ShapeDtypeStruct((P, 1, L), jnp.float32),
    )(qt, ks)


def _lane0_mask():
    return jnp.arange(16, dtype=jnp.int32) == 0


def _store_scalar(ref, pos, val):
    idx = jnp.full((16,), pos, dtype=jnp.int32)
    x = jnp.full((16,), val, dtype=ref.dtype)
    plsc.store_scatter(ref, [idx], x, mask=_lane0_mask())


def _sc_body(m_hbm, idx_hbm, row_v, cmax_v, idx_v,
             *, seq_len, n_top, lblk, pairs_per_subcore):
    n_chunks = seq_len // 16
    cvecs = n_chunks // 16
    wid = lax.axis_index("s") * 2 + lax.axis_index("c")

    for p in range(pairs_per_subcore):
        pair = wid * pairs_per_subcore + p
        pltpu.sync_copy(m_hbm.at[pair, 0], row_v)

        def _init(j, carry):
            v = row_v[pl.ds(j * 16, 16)]
            _store_scalar(cmax_v, j, jnp.max(v))
            return carry
        lax.fori_loop(0, n_chunks, _init, 0)

        for t in range(n_top // 16, 8):
            idx_v[pl.ds(t * 16, 16)] = jnp.zeros((16,), jnp.int32)

        def _extract(t, carry):
            rm = cmax_v[pl.ds(0, 16)]
            for j in range(1, cvecs):
                rm = jnp.maximum(rm, cmax_v[pl.ds(j * 16, 16)])
            gmax = jnp.max(rm)

            def _find(j, best):
                mj = cmax_v[pl.ds(j * 16, 16)] == gmax
                fj = plsc.all_reduce_ffs(mj)
                fj = jnp.asarray(fj).reshape(-1)[0]
                cand = j * 16 + fj
                hit = (fj < 16) & (best >= n_chunks)
                return jnp.where(hit, cand, best)
            chunk = lax.fori_loop(0, cvecs, _find, jnp.int32(n_chunks))

            v = row_v[pl.ds(chunk * 16, 16)]
            lane = plsc.all_reduce_ffs(v == gmax)
            lane = jnp.asarray(lane).reshape(-1)[0]
            elem = chunk * 16 + lane
            vrow = (elem % lblk) * (seq_len // lblk) + elem // lblk
            _store_scalar(idx_v, t, vrow)
            v2 = jnp.where(jnp.arange(16, dtype=jnp.int32) == lane, _NEG, v)
            row_v[pl.ds(chunk * 16, 16)] = v2
            _store_scalar(cmax_v, chunk, jnp.max(v2))
            return carry
        lax.fori_loop(0, n_top, _extract, 0)

        pltpu.sync_copy(idx_v, idx_hbm.at[pair])


def _sc_topk(m, lblk):
    P, _, L = m.shape
    U = 40
    pps = P // 32
    mesh = plsc.VectorSubcoreMesh(core_axis_name="c", subcore_axis_name="s")
    fn = pl.kernel(
        functools.partial(_sc_body, seq_len=L, n_top=U, lblk=lblk,
                          pairs_per_subcore=pps),
        out_type=jax.ShapeDtypeStruct((P, 128), jnp.int32),
        mesh=mesh,
        compiler_params=pltpu.CompilerParams(needs_layout_passes=False),
        scratch_types=[
            pltpu.VMEM((L,), jnp.float32),
            pltpu.VMEM((L // 16,), jnp.float32),
            pltpu.VMEM((128,), jnp.int32),
        ],
    )
    return fn(m)


def _tree_sum(xs):
    while len(xs) > 1:
        nxt = [xs[i] + xs[i + 1] for i in range(0, len(xs) - 1, 2)]
        if len(xs) % 2:
            nxt.append(xs[-1])
        xs = nxt
    return xs[0]


def _stage2_body(kt_ref, vt_ref, qt_ref, idx_ref, out_ref, *, scale, n_top):
    HV = kt_ref.shape[1]
    D = kt_ref.shape[2]
    LBLK = kt_ref.shape[3]
    L = HV * LBLK
    idxv = idx_ref[0, 0][:n_top]
    hh = idxv % HV
    ll = idxv // HV

    oh_h = (lax.broadcasted_iota(jnp.int32, (n_top, HV), 1)
            == hh[:, None]).astype(jnp.float32)
    oh_l = (lax.broadcasted_iota(jnp.int32, (n_top, LBLK), 1)
            == ll[:, None]).astype(jnp.float32)

    qt2 = qt_ref[0].reshape(HV * D, LBLK)
    G = lax.dot_general(oh_l, qt2, (((1,), (1,)), ((), ())),
                        preferred_element_type=jnp.float32)
    qr = _tree_sum([oh_h[:, h][:, None] * G[:, h * D:(h + 1) * D]
                    for h in range(HV)])
    qr = qr * scale

    S = jnp.concatenate(
        [lax.dot_general(qr, kt_ref[0, h], (((1,), (0,)), ((), ())),
                         preferred_element_type=jnp.float32)
         for h in range(HV)], axis=1)
    mx = jnp.max(S, axis=1, keepdims=True)
    E = jnp.exp(S - mx)
    den = jnp.sum(E, axis=1, keepdims=True)
    A = E / den
    upd = _tree_sum(
        [lax.dot_general(A[:, h * LBLK:(h + 1) * LBLK], vt_ref[0, h],
                         (((1,), (1,)), ((), ())),
                         preferred_element_type=jnp.float32)
         for h in range(HV)])
    vsum = _tree_sum([jnp.sum(vt_ref[0, h], axis=1)[None, :]
                      for h in range(HV)])

    oh2 = (lax.broadcasted_iota(jnp.int32, (n_top, L), 1)
           == idxv[:, None]).astype(jnp.float32)
    outT = lax.dot_general(upd - vsum, oh2, (((0,), (0,)), ((), ())),
                           preferred_element_type=jnp.float32)
    out_ref[0, 0] = outT + jnp.broadcast_to(vsum.reshape(D, 1), (D, L))


def _stage2(kt, vt, qt, idx3):
    B, HV, D, L = kt.shape
    P = B * HV
    LBLK = L // HV
    U = 40
    scale = 1.0 / np.sqrt(64)
    return pl.pallas_call(
        functools.partial(_stage2_body, scale=scale, n_top=U),
        grid=(P,),
        in_specs=[
            pl.BlockSpec((1, HV, D, LBLK), lambda i: (i // 16, 0, 0, i % 16)),
            pl.BlockSpec((1, HV, D, LBLK), lambda i: (i // 16, 0, 0, i % 16)),
            pl.BlockSpec((1, HV, D, LBLK), lambda i: (i // 16, 0, 0, i % 16)),
            pl.BlockSpec((1, 1, 128), lambda i: (i, 0, 0)),
        ],
        out_specs=pl.BlockSpec((1, 1, D, L), lambda i: (i // 16, i % 16, 0, 0)),
        out_shape=jax.ShapeDtypeStruct((B, HV, D, L), jnp.float32),
    )(kt, vt, qt, idx3)


def kernel(queries, keys, values):
    B, L, H, D = queries.shape
    S = keys.shape[1]
    HV = H * D // 64
    LBLK = L // HV
    P = B * HV

    qt = jnp.transpose(queries, (0, 2, 3, 1))
    kt = jnp.transpose(keys, (0, 2, 3, 1))
    vt = jnp.transpose(values, (0, 2, 3, 1))

    u = _FACTOR * int(np.ceil(np.log(L)))
    cols = [
        lax.slice(kt, (0, int(s) % HV, 0, int(s) // HV),
                  (B, int(s) % HV + 1, 64, L), (1, 1, 1, LBLK))
        for s in _SAMP[:u]
    ]
    ks = jnp.concatenate(cols, axis=1)
    ks = jnp.transpose(ks, (0, 3, 1, 2)).reshape(P, u, 64)

    m = _stage1(qt, ks)
    idx = _sc_topk(m, LBLK)
    ctx = _stage2(kt, vt, qt, idx.reshape(P, 1, 128))
    return jnp.transpose(ctx, (0, 1, 3, 2))


# speedup vs baseline: 1.6747x; 1.6747x over previous
import functools

import jax
import jax.numpy as jnp
from jax import lax
from jax.experimental import pallas as pl
from jax.experimental.pallas import tpu as pltpu
from jax.experimental.pallas import tpu_sc as plsc

import numpy as np

_FACTOR = 5
_NEG = np.float32(-3.0e38)

_SAMP = np.array([1220, 18, 1207, 1217, 653, 1387, 385, 295, 6, 1282, 552,
                  2034, 1433, 475, 1996, 1810, 1611, 898, 835, 519, 1590,
                  651, 268, 1731, 1132, 1553, 1008, 539, 284, 1335, 261,
                  676, 1493, 46, 1075, 20, 814, 1970, 1873, 2029],
                 dtype=np.int32)


def _stage1_body(qt_ref, ks_ref, m_ref, *, seq_len, n_heads):
    ksb = ks_ref[0]
    inv = 1.0 / seq_len
    for h in range(n_heads):
        qt_h = qt_ref[0, h]
        s = lax.dot_general(ksb, qt_h, (((1,), (0,)), ((), ())),
                            preferred_element_type=jnp.float32)
        m_ref[0, 0, pl.ds(h * qt_h.shape[1], qt_h.shape[1])] = (
            jnp.max(s, axis=0) - jnp.sum(s, axis=0) * inv)


def _stage1(qt, ks):
    B, HV, D, L = qt.shape
    P = B * HV
    U = ks.shape[1]
    LBLK = L // HV
    return pl.pallas_call(
        functools.partial(_stage1_body, seq_len=L, n_heads=HV),
        grid=(P,),
        in_specs=[
            pl.BlockSpec((1, HV, D, LBLK), lambda i: (i // 16, 0, 0, i % 16)),
            pl.BlockSpec((1, U, D), lambda i: (i, 0, 0)),
        ],
        out_specs=pl.BlockSpec((1, 1, L), lambda i: (i, 0, 0)),
        out_shape=jax.ShapeDtypeStruct((P, 1, L), jnp.float32),
    )(qt, ks)


def _lane0_mask():
    return jnp.arange(16, dtype=jnp.int32) == 0


def _store_scalar(ref, pos, val):
    idx = jnp.full((16,), pos, dtype=jnp.int32)
    x = jnp.full((16,), val, dtype=ref.dtype)
    plsc.store_scatter(ref, [idx], x, mask=_lane0_mask())


def _sc_body(m_hbm, idx_hbm, row_v, cmax_v, idx_v,
             *, seq_len, n_top, lblk, pairs_per_subcore):
    n_chunks = seq_len // 16
    cvecs = n_chunks // 16
    wid = lax.axis_index("s") * 2 + lax.axis_index("c")

    for p in range(pairs_per_subcore):
        pair = wid * pairs_per_subcore + p
        pltpu.sync_copy(m_hbm.at[pair, 0], row_v)

        def _init(j, carry):
            v = row_v[pl.ds(j * 16, 16)]
            _store_scalar(cmax_v, j, jnp.max(v))
            return carry
        lax.fori_loop(0, n_chunks, _init, 0)

        for t in range(n_top // 16, 8):
            idx_v[pl.ds(t * 16, 16)] = jnp.zeros((16,), jnp.int32)

        def _extract(t, carry):
            rm = cmax_v[pl.ds(0, 16)]
            for j in range(1, cvecs):
                rm = jnp.maximum(rm, cmax_v[pl.ds(j * 16, 16)])
            gmax = jnp.max(rm)

            def _find(j, best):
                mj = cmax_v[pl.ds(j * 16, 16)] == gmax
                fj = plsc.all_reduce_ffs(mj)
                fj = jnp.asarray(fj).reshape(-1)[0]
                cand = j * 16 + fj
                hit = (fj < 16) & (best >= n_chunks)
                return jnp.where(hit, cand, best)
            chunk = lax.fori_loop(0, cvecs, _find, jnp.int32(n_chunks))

            v = row_v[pl.ds(chunk * 16, 16)]
            lane = plsc.all_reduce_ffs(v == gmax)
            lane = jnp.asarray(lane).reshape(-1)[0]
            elem = chunk * 16 + lane
            vrow = (elem % lblk) * (seq_len // lblk) + elem // lblk
            _store_scalar(idx_v, t, vrow)
            v2 = jnp.where(jnp.arange(16, dtype=jnp.int32) == lane, _NEG, v)
            row_v[pl.ds(chunk * 16, 16)] = v2
            _store_scalar(cmax_v, chunk, jnp.max(v2))
            return carry
        lax.fori_loop(0, n_top, _extract, 0)

        pltpu.sync_copy(idx_v, idx_hbm.at[pair])


def _sc_topk(m, lblk):
    P, _, L = m.shape
    U = 40
    pps = P // 32
    mesh = plsc.VectorSubcoreMesh(core_axis_name="c", subcore_axis_name="s")
    fn = pl.kernel(
        functools.partial(_sc_body, seq_len=L, n_top=U, lblk=lblk,
                          pairs_per_subcore=pps),
        out_type=jax.ShapeDtypeStruct((P, 128), jnp.int32),
        mesh=mesh,
        compiler_params=pltpu.CompilerParams(needs_layout_passes=False),
        scratch_types=[
            pltpu.VMEM((L,), jnp.float32),
            pltpu.VMEM((L // 16,), jnp.float32),
            pltpu.VMEM((128,), jnp.int32),
        ],
    )
    return fn(m)


def _tree_sum(xs):
    while len(xs) > 1:
        nxt = [xs[i] + xs[i + 1] for i in range(0, len(xs) - 1, 2)]
        if len(xs) % 2:
            nxt.append(xs[-1])
        xs = nxt
    return xs[0]


def _stage2_body(kt_ref, vt_ref, qt_ref, idx_ref, out_ref, *, scale, n_top):
    HV = kt_ref.shape[1]
    D = kt_ref.shape[2]
    LBLK = kt_ref.shape[3]
    L = HV * LBLK
    idxv = idx_ref[0, 0][:n_top]
    hh = idxv % HV
    ll = idxv // HV

    oh_h = (lax.broadcasted_iota(jnp.int32, (n_top, HV), 1)
            == hh[:, None]).astype(jnp.float32)
    oh_l = (lax.broadcasted_iota(jnp.int32, (n_top, LBLK), 1)
            == ll[:, None]).astype(jnp.float32)

    qt2 = qt_ref[0].reshape(HV * D, LBLK)
    G = lax.dot_general(oh_l, qt2, (((1,), (1,)), ((), ())),
                        preferred_element_type=jnp.float32)
    qr = _tree_sum([oh_h[:, h][:, None] * G[:, h * D:(h + 1) * D]
                    for h in range(HV)])
    qr = qr * scale

    S = jnp.concatenate(
        [lax.dot_general(qr, kt_ref[0, h], (((1,), (0,)), ((), ())),
                         preferred_element_type=jnp.float32)
         for h in range(HV)], axis=1)
    mx = jnp.max(S, axis=1, keepdims=True)
    E = jnp.exp(S - mx)
    den = jnp.sum(E, axis=1, keepdims=True)
    A = E / den
    upd = _tree_sum(
        [lax.dot_general(A[:, h * LBLK:(h + 1) * LBLK], vt_ref[0, h],
                         (((1,), (1,)), ((), ())),
                         preferred_element_type=jnp.float32)
         for h in range(HV)])
    vsum = _tree_sum([jnp.sum(vt_ref[0, h], axis=1)[None, :]
                      for h in range(HV)])

    oh2 = (lax.broadcasted_iota(jnp.int32, (n_top, L), 1)
           == idxv[:, None]).astype(jnp.float32)
    outT = lax.dot_general(upd - vsum, oh2, (((0,), (0,)), ((), ())),
                           preferred_element_type=jnp.float32)
    out_ref[0, 0] = outT + jnp.broadcast_to(vsum.reshape(D, 1), (D, L))


def _stage2(kt, vt, qt, idx3):
    B, HV, D, L = kt.shape
    P = B * HV
    LBLK = L // HV
    U = 40
    scale = 1.0 / np.sqrt(64)
    return pl.pallas_call(
        functools.partial(_stage2_body, scale=scale, n_top=U),
        grid=(P,),
        in_specs=[
            pl.BlockSpec((1, HV, D, LBLK), lambda i: (i // 16, 0, 0, i % 16)),
            pl.BlockSpec((1, HV, D, LBLK), lambda i: (i // 16, 0, 0, i % 16)),
            pl.BlockSpec((1, HV, D, LBLK), lambda i: (i // 16, 0, 0, i % 16)),
            pl.BlockSpec((1, 1, 128), lambda i: (i, 0, 0)),
        ],
        out_specs=pl.BlockSpec((1, 1, D, L), lambda i: (i // 16, i % 16, 0, 0)),
        out_shape=jax.ShapeDtypeStruct((B, HV, D, L), jnp.float32),
    )(kt, vt, qt, idx3)


def kernel(queries, keys, values):
    B, L, H, D = queries.shape
    S = keys.shape[1]
    HV = H * D // 64
    LBLK = L // HV
    P = B * HV

    qt = jnp.transpose(queries, (0, 2, 3, 1))
    kt = jnp.transpose(keys, (0, 2, 3, 1))
    vt = jnp.transpose(values, (0, 2, 3, 1))

    u = _FACTOR * int(np.ceil(np.log(L)))
    sh = _SAMP[:u] % HV
    sl = _SAMP[:u] // HV
    lidx = np.arange(HV, dtype=np.int32)[:, None] * LBLK + sl[None, :]
    ks = kt[:, jnp.asarray(sh)[None, :], :, jnp.asarray(lidx)]
    ks = jnp.transpose(ks, (2, 0, 1, 3)).reshape(P, u, 64)

    m = _stage1(qt, ks)
    idx = _sc_topk(m, LBLK)
    ctx = _stage2(kt, vt, qt, idx.reshape(P, 1, 128))
    return jnp.transpose(ctx, (0, 1, 3, 2))


# speedup vs baseline: 1.8984x; 1.1335x over previous
import functools

import jax
import jax.numpy as jnp
from jax import lax
from jax.experimental import pallas as pl
from jax.experimental.pallas import tpu as pltpu
from jax.experimental.pallas import tpu_sc as plsc

import numpy as np

_FACTOR = 5
_NEG = np.float32(-3.0e38)

_SAMP = np.array([1220, 18, 1207, 1217, 653, 1387, 385, 295, 6, 1282, 552,
                  2034, 1433, 475, 1996, 1810, 1611, 898, 835, 519, 1590,
                  651, 268, 1731, 1132, 1553, 1008, 539, 284, 1335, 261,
                  676, 1493, 46, 1075, 20, 814, 1970, 1873, 2029],
                 dtype=np.int32)


def _tree_sum(xs):
    while len(xs) > 1:
        nxt = [xs[i] + xs[i + 1] for i in range(0, len(xs) - 1, 2)]
        if len(xs) % 2:
            nxt.append(xs[-1])
        xs = nxt
    return xs[0]


def _stage1_body(qt_ref, kt_ref, ohh_ref, ohl_ref, m_ref,
                 *, seq_len, n_heads, n_top):
    HV = n_heads
    D = qt_ref.shape[2]
    LBLK = qt_ref.shape[3]
    inv = 1.0 / seq_len
    oh_h = ohh_ref[...]
    oh_l = ohl_ref[...]
    kt2 = kt_ref[0].reshape(HV * D, LBLK)
    Gk = lax.dot_general(oh_l, kt2, (((1,), (1,)), ((), ())),
                         preferred_element_type=jnp.float32)
    ksb = _tree_sum([oh_h[:, h][:, None] * Gk[:, h * D:(h + 1) * D]
                     for h in range(HV)])
    for h in range(HV):
        qt_h = qt_ref[0, h]
        s = lax.dot_general(ksb, qt_h, (((1,), (0,)), ((), ())),
                            preferred_element_type=jnp.float32)
        m_ref[0, 0, pl.ds(h * LBLK, LBLK)] = (
            jnp.max(s, axis=0) - jnp.sum(s, axis=0) * inv)


def _stage1(qt, kt):
    B, HV, D, L = qt.shape
    P = B * HV
    LBLK = L // HV
    U = 40
    ohh = jnp.asarray(np.equal(np.arange(HV)[None, :],
                               (_SAMP[:U] % HV)[:, None]).astype(np.float32))
    ohl = jnp.asarray(np.equal(np.arange(LBLK)[None, :],
                               (_SAMP[:U] // HV)[:, None]).astype(np.float32))
    return pl.pallas_call(
        functools.partial(_stage1_body, seq_len=L, n_heads=HV, n_top=U),
        grid=(P,),
        in_specs=[
            pl.BlockSpec((1, HV, D, LBLK), lambda i: (i // 16, 0, 0, i % 16)),
            pl.BlockSpec((1, HV, D, LBLK), lambda i: (i // 16, 0, 0, i % 16)),
            pl.BlockSpec((U, HV), lambda i: (0, 0)),
            pl.BlockSpec((U, LBLK), lambda i: (0, 0)),
        ],
        out_specs=pl.BlockSpec((1, 1, L), lambda i: (i, 0, 0)),
        out_shape=jax.ShapeDtypeStruct((P, 1, L), jnp.float32),
    )(qt, kt, ohh, ohl)


def _lane0_mask():
    return jnp.arange(16, dtype=jnp.int32) == 0


def _store_scalar(ref, pos, val):
    idx = jnp.full((16,), pos, dtype=jnp.int32)
    x = jnp.full((16,), val, dtype=ref.dtype)
    plsc.store_scatter(ref, [idx], x, mask=_lane0_mask())


def _sc_body(m_hbm, idx_hbm, row_v, cmax_v, idx_v,
             *, seq_len, n_top, lblk, pairs_per_subcore):
    n_chunks = seq_len // 16
    cvecs = n_chunks // 16
    wid = lax.axis_index("s") * 2 + lax.axis_index("c")

    for p in range(pairs_per_subcore):
        pair = wid * pairs_per_subcore + p
        pltpu.sync_copy(m_hbm.at[pair, 0], row_v)

        def _init(j, carry):
            v = row_v[pl.ds(j * 16, 16)]
            _store_scalar(cmax_v, j, jnp.max(v))
            return carry
        lax.fori_loop(0, n_chunks, _init, 0)

        for t in range(n_top // 16, 8):
            idx_v[pl.ds(t * 16, 16)] = jnp.zeros((16,), jnp.int32)

        def _extract(t, carry):
            rm = cmax_v[pl.ds(0, 16)]
            for j in range(1, cvecs):
                rm = jnp.maximum(rm, cmax_v[pl.ds(j * 16, 16)])
            gmax = jnp.max(rm)

            def _find(j, best):
                mj = cmax_v[pl.ds(j * 16, 16)] == gmax
                fj = plsc.all_reduce_ffs(mj)
                fj = jnp.asarray(fj).reshape(-1)[0]
                cand = j * 16 + fj
                hit = (fj < 16) & (best >= n_chunks)
                return jnp.where(hit, cand, best)
            chunk = lax.fori_loop(0, cvecs, _find, jnp.int32(n_chunks))

            v = row_v[pl.ds(chunk * 16, 16)]
            lane = plsc.all_reduce_ffs(v == gmax)
            lane = jnp.asarray(lane).reshape(-1)[0]
            elem = chunk * 16 + lane
            vrow = (elem % lblk) * (seq_len // lblk) + elem // lblk
            _store_scalar(idx_v, t, vrow)
            v2 = jnp.where(jnp.arange(16, dtype=jnp.int32) == lane, _NEG, v)
            row_v[pl.ds(chunk * 16, 16)] = v2
            _store_scalar(cmax_v, chunk, jnp.max(v2))
            return carry
        lax.fori_loop(0, n_top, _extract, 0)

        pltpu.sync_copy(idx_v, idx_hbm.at[pair])


def _sc_topk(m, lblk):
    P, _, L = m.shape
    U = 40
    pps = P // 32
    mesh = plsc.VectorSubcoreMesh(core_axis_name="c", subcore_axis_name="s")
    fn = pl.kernel(
        functools.partial(_sc_body, seq_len=L, n_top=U, lblk=lblk,
                          pairs_per_subcore=pps),
        out_type=jax.ShapeDtypeStruct((P, 128), jnp.int32),
        mesh=mesh,
        compiler_params=pltpu.CompilerParams(needs_layout_passes=False),
        scratch_types=[
            pltpu.VMEM((L,), jnp.float32),
            pltpu.VMEM((L // 16,), jnp.float32),
            pltpu.VMEM((128,), jnp.int32),
        ],
    )
    return fn(m)


def _stage2_body(kt_ref, vt_ref, qt_ref, idx_ref, out_ref, *, scale, n_top):
    HV = kt_ref.shape[1]
    D = kt_ref.shape[2]
    LBLK = kt_ref.shape[3]
    L = HV * LBLK
    idxv = idx_ref[0, 0][:n_top]
    hh = idxv % HV
    ll = idxv // HV

    oh_h = (lax.broadcasted_iota(jnp.int32, (n_top, HV), 1)
            == hh[:, None]).astype(jnp.float32)
    oh_l = (lax.broadcasted_iota(jnp.int32, (n_top, LBLK), 1)
            == ll[:, None]).astype(jnp.float32)

    qt2 = qt_ref[0].reshape(HV * D, LBLK)
    G = lax.dot_general(oh_l, qt2, (((1,), (1,)), ((), ())),
                        preferred_element_type=jnp.float32)
    qr = _tree_sum([oh_h[:, h][:, None] * G[:, h * D:(h + 1) * D]
                    for h in range(HV)])
    qr = (qr * scale).astype(jnp.bfloat16)

    S = jnp.concatenate(
        [lax.dot_general(qr, kt_ref[0, h].astype(jnp.bfloat16),
                         (((1,), (0,)), ((), ())),
                         preferred_element_type=jnp.float32)
         for h in range(HV)], axis=1)
    mx = jnp.max(S, axis=1, keepdims=True)
    E = jnp.exp(S - mx)
    den = jnp.sum(E, axis=1, keepdims=True)
    A = (E / den).astype(jnp.bfloat16)
    upd = _tree_sum(
        [lax.dot_general(A[:, h * LBLK:(h + 1) * LBLK],
                         vt_ref[0, h].astype(jnp.bfloat16),
                         (((1,), (1,)), ((), ())),
                         preferred_element_type=jnp.float32)
         for h in range(HV)])
    vsum = _tree_sum([jnp.sum(vt_ref[0, h], axis=1)[None, :]
                      for h in range(HV)])

    oh2 = (lax.broadcasted_iota(jnp.int32, (n_top, L), 1)
           == idxv[:, None]).astype(jnp.float32)
    outT = lax.dot_general(upd - vsum, oh2, (((0,), (0,)), ((), ())),
                           preferred_element_type=jnp.float32)
    out_ref[0, 0] = outT + jnp.broadcast_to(vsum.reshape(D, 1), (D, L))


def _stage2(kt, vt, qt, idx3):
    B, HV, D, L = kt.shape
    P = B * HV
    LBLK = L // HV
    U = 40
    scale = 1.0 / np.sqrt(64)
    return pl.pallas_call(
        functools.partial(_stage2_body, scale=scale, n_top=U),
        grid=(P,),
        in_specs=[
            pl.BlockSpec((1, HV, D, LBLK), lambda i: (i // 16, 0, 0, i % 16)),
            pl.BlockSpec((1, HV, D, LBLK), lambda i: (i // 16, 0, 0, i % 16)),
            pl.BlockSpec((1, HV, D, LBLK), lambda i: (i // 16, 0, 0, i % 16)),
            pl.BlockSpec((1, 1, 128), lambda i: (i, 0, 0)),
        ],
        out_specs=pl.BlockSpec((1, 1, D, L), lambda i: (i // 16, i % 16, 0, 0)),
        out_shape=jax.ShapeDtypeStruct((B, HV, D, L), jnp.float32),
    )(kt, vt, qt, idx3)


def kernel(queries, keys, values):
    B, L, H, D = queries.shape
    S = keys.shape[1]
    HV = H * D // 64
    LBLK = L // HV
    P = B * HV

    qt = jnp.transpose(queries, (0, 2, 3, 1))
    kt = jnp.transpose(keys, (0, 2, 3, 1))
    vt = jnp.transpose(values, (0, 2, 3, 1))

    m = _stage1(qt, kt)
    idx = _sc_topk(m, LBLK)
    ctx = _stage2(kt, vt, qt, idx.reshape(P, 1, 128))
    return jnp.transpose(ctx, (0, 1, 3, 2))


# speedup vs baseline: 2.2508x; 1.1857x over previous
import functools

import jax
import jax.numpy as jnp
from jax import lax
from jax.experimental import pallas as pl
from jax.experimental.pallas import tpu as pltpu
from jax.experimental.pallas import tpu_sc as plsc

import numpy as np

_FACTOR = 5
_NEG = np.float32(-3.0e38)

_SAMP = np.array([1220, 18, 1207, 1217, 653, 1387, 385, 295, 6, 1282, 552,
                  2034, 1433, 475, 1996, 1810, 1611, 898, 835, 519, 1590,
                  651, 268, 1731, 1132, 1553, 1008, 539, 284, 1335, 261,
                  676, 1493, 46, 1075, 20, 814, 1970, 1873, 2029],
                 dtype=np.int32)


def _tree_sum(xs):
    while len(xs) > 1:
        nxt = [xs[i] + xs[i + 1] for i in range(0, len(xs) - 1, 2)]
        if len(xs) % 2:
            nxt.append(xs[-1])
        xs = nxt
    return xs[0]


def _stage1_body(qt_ref, kt_ref, ohh_ref, ohl_ref, m_ref,
                 *, seq_len, n_heads, n_top, ppb):
    HV = n_heads
    D = qt_ref.shape[2]
    LBLK = qt_ref.shape[3] // ppb
    inv = 1.0 / seq_len
    oh_h = ohh_ref[...]
    oh_l = ohl_ref[...]
    for p in range(ppb):
        kt2 = kt_ref[0][:, :, p * LBLK:(p + 1) * LBLK].reshape(HV * D, LBLK)
        Gk = lax.dot_general(oh_l, kt2, (((1,), (1,)), ((), ())),
                             preferred_element_type=jnp.float32)
        ksb = _tree_sum([oh_h[:, h][:, None] * Gk[:, h * D:(h + 1) * D]
                         for h in range(HV)])
        for h in range(HV):
            qt_h = qt_ref[0, h, :, p * LBLK:(p + 1) * LBLK]
            s = lax.dot_general(ksb, qt_h, (((1,), (0,)), ((), ())),
                                preferred_element_type=jnp.float32)
            m_ref[p, 0, pl.ds(h * LBLK, LBLK)] = (
                jnp.max(s, axis=0) - jnp.sum(s, axis=0) * inv)


def _stage1(qt, kt):
    B, HV, D, L = qt.shape
    P = B * HV
    LBLK = L // HV
    U = 40
    PPB = 2
    NB = HV // PPB
    ohh = jnp.asarray(np.equal(np.arange(HV)[None, :],
                               (_SAMP[:U] % HV)[:, None]).astype(np.float32))
    ohl = jnp.asarray(np.equal(np.arange(LBLK)[None, :],
                               (_SAMP[:U] // HV)[:, None]).astype(np.float32))
    return pl.pallas_call(
        functools.partial(_stage1_body, seq_len=L, n_heads=HV, n_top=U,
                          ppb=PPB),
        grid=(P // PPB,),
        in_specs=[
            pl.BlockSpec((1, HV, D, PPB * LBLK),
                         lambda i: (i // NB, 0, 0, i % NB)),
            pl.BlockSpec((1, HV, D, PPB * LBLK),
                         lambda i: (i // NB, 0, 0, i % NB)),
            pl.BlockSpec((U, HV), lambda i: (0, 0)),
            pl.BlockSpec((U, LBLK), lambda i: (0, 0)),
        ],
        out_specs=pl.BlockSpec((PPB, 1, L), lambda i: (i, 0, 0)),
        out_shape=jax.ShapeDtypeStruct((P, 1, L), jnp.float32),
    )(qt, kt, ohh, ohl)


def _lane0_mask():
    return jnp.arange(16, dtype=jnp.int32) == 0


def _store_scalar(ref, pos, val):
    idx = jnp.full((16,), pos, dtype=jnp.int32)
    x = jnp.full((16,), val, dtype=ref.dtype)
    plsc.store_scatter(ref, [idx], x, mask=_lane0_mask())


def _sc_body(m_hbm, idx_hbm, row_v, cmax_v, idx_v,
             *, seq_len, n_top, lblk, pairs_per_subcore):
    n_chunks = seq_len // 16
    cvecs = n_chunks // 16
    wid = lax.axis_index("s") * 2 + lax.axis_index("c")

    for p in range(pairs_per_subcore):
        pair = wid * pairs_per_subcore + p
        pltpu.sync_copy(m_hbm.at[pair, 0], row_v)

        def _init(j, carry):
            v = row_v[pl.ds(j * 16, 16)]
            _store_scalar(cmax_v, j, jnp.max(v))
            return carry
        lax.fori_loop(0, n_chunks, _init, 0)

        for t in range(n_top // 16, 8):
            idx_v[pl.ds(t * 16, 16)] = jnp.zeros((16,), jnp.int32)

        def _extract(t, carry):
            rm = cmax_v[pl.ds(0, 16)]
            for j in range(1, cvecs):
                rm = jnp.maximum(rm, cmax_v[pl.ds(j * 16, 16)])
            gmax = jnp.max(rm)

            def _find(j, best):
                mj = cmax_v[pl.ds(j * 16, 16)] == gmax
                fj = plsc.all_reduce_ffs(mj)
                fj = jnp.asarray(fj).reshape(-1)[0]
                cand = j * 16 + fj
                hit = (fj < 16) & (best >= n_chunks)
                return jnp.where(hit, cand, best)
            chunk = lax.fori_loop(0, cvecs, _find, jnp.int32(n_chunks))

            v = row_v[pl.ds(chunk * 16, 16)]
            lane = plsc.all_reduce_ffs(v == gmax)
            lane = jnp.asarray(lane).reshape(-1)[0]
            elem = chunk * 16 + lane
            vrow = (elem % lblk) * (seq_len // lblk) + elem // lblk
            _store_scalar(idx_v, t, vrow)
            v2 = jnp.where(jnp.arange(16, dtype=jnp.int32) == lane, _NEG, v)
            row_v[pl.ds(chunk * 16, 16)] = v2
            _store_scalar(cmax_v, chunk, jnp.max(v2))
            return carry
        lax.fori_loop(0, n_top, _extract, 0)

        pltpu.sync_copy(idx_v, idx_hbm.at[pair])


def _sc_topk(m, lblk):
    P, _, L = m.shape
    U = 40
    pps = P // 32
    mesh = plsc.VectorSubcoreMesh(core_axis_name="c", subcore_axis_name="s")
    fn = pl.kernel(
        functools.partial(_sc_body, seq_len=L, n_top=U, lblk=lblk,
                          pairs_per_subcore=pps),
        out_type=jax.ShapeDtypeStruct((P, 128), jnp.int32),
        mesh=mesh,
        compiler_params=pltpu.CompilerParams(needs_layout_passes=False),
        scratch_types=[
            pltpu.VMEM((L,), jnp.float32),
            pltpu.VMEM((L // 16,), jnp.float32),
            pltpu.VMEM((128,), jnp.int32),
        ],
    )
    return fn(m)


def _stage2_body(kt_ref, vt_ref, qt_ref, idx_ref, out_ref,
                 *, scale, n_top, ppb):
    HV = kt_ref.shape[1]
    D = kt_ref.shape[2]
    LBLK = kt_ref.shape[3] // ppb
    L = HV * LBLK
    for p in range(ppb):
        idxv = idx_ref[p, 0][:n_top]
        hh = idxv % HV
        ll = idxv // HV

        oh_h = (lax.broadcasted_iota(jnp.int32, (n_top, HV), 1)
                == hh[:, None]).astype(jnp.float32)
        oh_l = (lax.broadcasted_iota(jnp.int32, (n_top, LBLK), 1)
                == ll[:, None]).astype(jnp.float32)

        qt2 = qt_ref[0][:, :, p * LBLK:(p + 1) * LBLK].reshape(HV * D, LBLK)
        G = lax.dot_general(oh_l, qt2, (((1,), (1,)), ((), ())),
                            preferred_element_type=jnp.float32)
        qr = _tree_sum([oh_h[:, h][:, None] * G[:, h * D:(h + 1) * D]
                        for h in range(HV)])
        qr = (qr * scale).astype(jnp.bfloat16)

        S = jnp.concatenate(
            [lax.dot_general(
                qr,
                kt_ref[0, h, :, p * LBLK:(p + 1) * LBLK].astype(jnp.bfloat16),
                (((1,), (0,)), ((), ())),
                preferred_element_type=jnp.float32)
             for h in range(HV)], axis=1)
        mx = jnp.max(S, axis=1, keepdims=True)
        E = jnp.exp(S - mx)
        den = jnp.sum(E, axis=1, keepdims=True)
        A = (E / den).astype(jnp.bfloat16)
        upd = _tree_sum(
            [lax.dot_general(
                A[:, h * LBLK:(h + 1) * LBLK],
                vt_ref[0, h, :, p * LBLK:(p + 1) * LBLK].astype(jnp.bfloat16),
                (((1,), (1,)), ((), ())),
                preferred_element_type=jnp.float32)
             for h in range(HV)])
        vsum = _tree_sum(
            [jnp.sum(vt_ref[0, h, :, p * LBLK:(p + 1) * LBLK], axis=1)[None, :]
             for h in range(HV)])

        oh2 = (lax.broadcasted_iota(jnp.int32, (n_top, L), 1)
               == idxv[:, None]).astype(jnp.float32)
        outT = lax.dot_general(upd - vsum, oh2, (((0,), (0,)), ((), ())),
                               preferred_element_type=jnp.float32)
        out_ref[0, p] = outT + jnp.broadcast_to(vsum.reshape(D, 1), (D, L))


def _stage2(kt, vt, qt, idx3):
    B, HV, D, L = kt.shape
    P = B * HV
    LBLK = L // HV
    U = 40
    PPB = 2
    NB = HV // PPB
    scale = 1.0 / np.sqrt(64)
    return pl.pallas_call(
        functools.partial(_stage2_body, scale=scale, n_top=U, ppb=PPB),
        grid=(P // PPB,),
        in_specs=[
            pl.BlockSpec((1, HV, D, PPB * LBLK),
                         lambda i: (i // NB, 0, 0, i % NB)),
            pl.BlockSpec((1, HV, D, PPB * LBLK),
                         lambda i: (i // NB, 0, 0, i % NB)),
            pl.BlockSpec((1, HV, D, PPB * LBLK),
                         lambda i: (i // NB, 0, 0, i % NB)),
            pl.BlockSpec((PPB, 1, 128), lambda i: (i, 0, 0)),
        ],
        out_specs=pl.BlockSpec((1, PPB, D, L),
                               lambda i: (i // NB, i % NB, 0, 0)),
        out_shape=jax.ShapeDtypeStruct((B, HV, D, L), jnp.float32),
    )(kt, vt, qt, idx3)


def kernel(queries, keys, values):
    B, L, H, D = queries.shape
    S = keys.shape[1]
    HV = H * D // 64
    LBLK = L // HV
    P = B * HV

    qt = jnp.transpose(queries, (0, 2, 3, 1))
    kt = jnp.transpose(keys, (0, 2, 3, 1))
    vt = jnp.transpose(values, (0, 2, 3, 1))

    m = _stage1(qt, kt)
    idx = _sc_topk(m, LBLK)
    ctx = _stage2(kt, vt, qt, idx.reshape(P, 1, 128))
    return jnp.transpose(ctx, (0, 1, 3, 2))


# speedup vs baseline: 2.3881x; 1.0610x over previous
import functools

import jax
import jax.numpy as jnp
from jax import lax
from jax.experimental import pallas as pl
from jax.experimental.pallas import tpu as pltpu
from jax.experimental.pallas import tpu_sc as plsc

import numpy as np

_FACTOR = 5
_NEG = np.float32(-3.0e38)

_SAMP = np.array([1220, 18, 1207, 1217, 653, 1387, 385, 295, 6, 1282, 552,
                  2034, 1433, 475, 1996, 1810, 1611, 898, 835, 519, 1590,
                  651, 268, 1731, 1132, 1553, 1008, 539, 284, 1335, 261,
                  676, 1493, 46, 1075, 20, 814, 1970, 1873, 2029],
                 dtype=np.int32)


def _tree_sum(xs):
    while len(xs) > 1:
        nxt = [xs[i] + xs[i + 1] for i in range(0, len(xs) - 1, 2)]
        if len(xs) % 2:
            nxt.append(xs[-1])
        xs = nxt
    return xs[0]


def _stage1_body(qt_ref, kt_ref, ohh_ref, ohl_ref, m_ref,
                 *, seq_len, n_heads, n_top, ppb):
    HV = n_heads
    D = qt_ref.shape[2]
    LBLK = qt_ref.shape[3] // ppb
    inv = 1.0 / seq_len
    oh_h = ohh_ref[...]
    oh_l = ohl_ref[...]
    for p in range(ppb):
        kt2 = kt_ref[0][:, :, p * LBLK:(p + 1) * LBLK].reshape(HV * D, LBLK)
        Gk = lax.dot_general(oh_l, kt2, (((1,), (1,)), ((), ())),
                             preferred_element_type=jnp.float32)
        ksb = _tree_sum([oh_h[:, h][:, None] * Gk[:, h * D:(h + 1) * D]
                         for h in range(HV)])
        for h in range(HV):
            qt_h = qt_ref[0, h, :, p * LBLK:(p + 1) * LBLK]
            s = lax.dot_general(ksb, qt_h, (((1,), (0,)), ((), ())),
                                preferred_element_type=jnp.float32)
            m_ref[p, 0, pl.ds(h * LBLK, LBLK)] = (
                jnp.max(s, axis=0) - jnp.sum(s, axis=0) * inv)


def _stage1(qt, kt):
    B, HV, D, L = qt.shape
    P = B * HV
    LBLK = L // HV
    U = 40
    PPB = 4
    NB = HV // PPB
    ohh = jnp.asarray(np.equal(np.arange(HV)[None, :],
                               (_SAMP[:U] % HV)[:, None]).astype(np.float32))
    ohl = jnp.asarray(np.equal(np.arange(LBLK)[None, :],
                               (_SAMP[:U] // HV)[:, None]).astype(np.float32))
    return pl.pallas_call(
        functools.partial(_stage1_body, seq_len=L, n_heads=HV, n_top=U,
                          ppb=PPB),
        grid=(P // PPB,),
        in_specs=[
            pl.BlockSpec((1, HV, D, PPB * LBLK),
                         lambda i: (i // NB, 0, 0, i % NB)),
            pl.BlockSpec((1, HV, D, PPB * LBLK),
                         lambda i: (i // NB, 0, 0, i % NB)),
            pl.BlockSpec((U, HV), lambda i: (0, 0)),
            pl.BlockSpec((U, LBLK), lambda i: (0, 0)),
        ],
        out_specs=pl.BlockSpec((PPB, 1, L), lambda i: (i, 0, 0)),
        out_shape=jax.ShapeDtypeStruct((P, 1, L), jnp.float32),
    )(qt, kt, ohh, ohl)


def _lane0_mask():
    return jnp.arange(16, dtype=jnp.int32) == 0


def _store_scalar(ref, pos, val):
    idx = jnp.full((16,), pos, dtype=jnp.int32)
    x = jnp.full((16,), val, dtype=ref.dtype)
    plsc.store_scatter(ref, [idx], x, mask=_lane0_mask())


def _sc_body(m_hbm, idx_hbm, row_v, cmax_v, idx_v,
             *, seq_len, n_top, lblk, pairs_per_subcore):
    n_chunks = seq_len // 16
    cvecs = n_chunks // 16
    wid = lax.axis_index("s") * 2 + lax.axis_index("c")

    for p in range(pairs_per_subcore):
        pair = wid * pairs_per_subcore + p
        pltpu.sync_copy(m_hbm.at[pair, 0], row_v)

        def _init(j, carry):
            v = row_v[pl.ds(j * 16, 16)]
            _store_scalar(cmax_v, j, jnp.max(v))
            return carry
        lax.fori_loop(0, n_chunks, _init, 0)

        for t in range(n_top // 16, 8):
            idx_v[pl.ds(t * 16, 16)] = jnp.zeros((16,), jnp.int32)

        def _extract(t, carry):
            rm = cmax_v[pl.ds(0, 16)]
            for j in range(1, cvecs):
                rm = jnp.maximum(rm, cmax_v[pl.ds(j * 16, 16)])
            gmax = jnp.max(rm)

            def _find(j, best):
                mj = cmax_v[pl.ds(j * 16, 16)] == gmax
                fj = plsc.all_reduce_ffs(mj)
                fj = jnp.asarray(fj).reshape(-1)[0]
                cand = j * 16 + fj
                hit = (fj < 16) & (best >= n_chunks)
                return jnp.where(hit, cand, best)
            chunk = lax.fori_loop(0, cvecs, _find, jnp.int32(n_chunks))

            v = row_v[pl.ds(chunk * 16, 16)]
            lane = plsc.all_reduce_ffs(v == gmax)
            lane = jnp.asarray(lane).reshape(-1)[0]
            elem = chunk * 16 + lane
            vrow = (elem % lblk) * (seq_len // lblk) + elem // lblk
            _store_scalar(idx_v, t, vrow)
            v2 = jnp.where(jnp.arange(16, dtype=jnp.int32) == lane, _NEG, v)
            row_v[pl.ds(chunk * 16, 16)] = v2
            _store_scalar(cmax_v, chunk, jnp.max(v2))
            return carry
        lax.fori_loop(0, n_top, _extract, 0)

        pltpu.sync_copy(idx_v, idx_hbm.at[pair])


def _sc_topk(m, lblk):
    P, _, L = m.shape
    U = 40
    pps = P // 32
    mesh = plsc.VectorSubcoreMesh(core_axis_name="c", subcore_axis_name="s")
    fn = pl.kernel(
        functools.partial(_sc_body, seq_len=L, n_top=U, lblk=lblk,
                          pairs_per_subcore=pps),
        out_type=jax.ShapeDtypeStruct((P, 128), jnp.int32),
        mesh=mesh,
        compiler_params=pltpu.CompilerParams(needs_layout_passes=False),
        scratch_types=[
            pltpu.VMEM((L,), jnp.float32),
            pltpu.VMEM((L // 16,), jnp.float32),
            pltpu.VMEM((128,), jnp.int32),
        ],
    )
    return fn(m)


def _stage2_body(kt_ref, vt_ref, qt_ref, idx_ref, out_ref,
                 *, scale, n_top, ppb):
    HV = kt_ref.shape[1]
    D = kt_ref.shape[2]
    LBLK = kt_ref.shape[3] // ppb
    L = HV * LBLK
    for p in range(ppb):
        idxv = idx_ref[p, 0][:n_top]
        hh = idxv % HV
        ll = idxv // HV

        oh_h = (lax.broadcasted_iota(jnp.int32, (n_top, HV), 1)
                == hh[:, None]).astype(jnp.float32)
        oh_l = (lax.broadcasted_iota(jnp.int32, (n_top, LBLK), 1)
                == ll[:, None]).astype(jnp.float32)

        qt2 = qt_ref[0][:, :, p * LBLK:(p + 1) * LBLK].reshape(HV * D, LBLK)
        G = lax.dot_general(oh_l, qt2, (((1,), (1,)), ((), ())),
                            preferred_element_type=jnp.float32)
        qr = _tree_sum([oh_h[:, h][:, None] * G[:, h * D:(h + 1) * D]
                        for h in range(HV)])
        qr = (qr * scale).astype(jnp.bfloat16)

        S = jnp.concatenate(
            [lax.dot_general(
                qr,
                kt_ref[0, h, :, p * LBLK:(p + 1) * LBLK].astype(jnp.bfloat16),
                (((1,), (0,)), ((), ())),
                preferred_element_type=jnp.float32)
             for h in range(HV)], axis=1)
        mx = jnp.max(S, axis=1, keepdims=True)
        E = jnp.exp(S - mx)
        den = jnp.sum(E, axis=1, keepdims=True)
        A = (E / den).astype(jnp.bfloat16)
        upd = _tree_sum(
            [lax.dot_general(
                A[:, h * LBLK:(h + 1) * LBLK],
                vt_ref[0, h, :, p * LBLK:(p + 1) * LBLK].astype(jnp.bfloat16),
                (((1,), (1,)), ((), ())),
                preferred_element_type=jnp.float32)
             for h in range(HV)])
        vsum = _tree_sum(
            [jnp.sum(vt_ref[0, h, :, p * LBLK:(p + 1) * LBLK], axis=1)[None, :]
             for h in range(HV)])

        oh2 = (lax.broadcasted_iota(jnp.int32, (n_top, L), 1)
               == idxv[:, None]).astype(jnp.float32)
        outT = lax.dot_general(upd - vsum, oh2, (((0,), (0,)), ((), ())),
                               preferred_element_type=jnp.float32)
        out_ref[0, p] = outT + jnp.broadcast_to(vsum.reshape(D, 1), (D, L))


def _stage2(kt, vt, qt, idx3):
    B, HV, D, L = kt.shape
    P = B * HV
    LBLK = L // HV
    U = 40
    PPB = 4
    NB = HV // PPB
    scale = 1.0 / np.sqrt(64)
    return pl.pallas_call(
        functools.partial(_stage2_body, scale=scale, n_top=U, ppb=PPB),
        grid=(P // PPB,),
        in_specs=[
            pl.BlockSpec((1, HV, D, PPB * LBLK),
                         lambda i: (i // NB, 0, 0, i % NB)),
            pl.BlockSpec((1, HV, D, PPB * LBLK),
                         lambda i: (i // NB, 0, 0, i % NB)),
            pl.BlockSpec((1, HV, D, PPB * LBLK),
                         lambda i: (i // NB, 0, 0, i % NB)),
            pl.BlockSpec((PPB, 1, 128), lambda i: (i, 0, 0)),
        ],
        out_specs=pl.BlockSpec((1, PPB, D, L),
                               lambda i: (i // NB, i % NB, 0, 0)),
        out_shape=jax.ShapeDtypeStruct((B, HV, D, L), jnp.float32),
    )(kt, vt, qt, idx3)


def kernel(queries, keys, values):
    B, L, H, D = queries.shape
    S = keys.shape[1]
    HV = H * D // 64
    LBLK = L // HV
    P = B * HV

    qt = jnp.transpose(queries, (0, 2, 3, 1))
    kt = jnp.transpose(keys, (0, 2, 3, 1))
    vt = jnp.transpose(values, (0, 2, 3, 1))

    m = _stage1(qt, kt)
    idx = _sc_topk(m, LBLK)
    ctx = _stage2(kt, vt, qt, idx.reshape(P, 1, 128))
    return jnp.transpose(ctx, (0, 1, 3, 2))


# speedup vs baseline: 2.4177x; 1.0124x over previous
import functools

import jax
import jax.numpy as jnp
from jax import lax
from jax.experimental import pallas as pl
from jax.experimental.pallas import tpu as pltpu
from jax.experimental.pallas import tpu_sc as plsc

import numpy as np

_FACTOR = 5
_NEG = np.float32(-3.0e38)

_SAMP = np.array([1220, 18, 1207, 1217, 653, 1387, 385, 295, 6, 1282, 552,
                  2034, 1433, 475, 1996, 1810, 1611, 898, 835, 519, 1590,
                  651, 268, 1731, 1132, 1553, 1008, 539, 284, 1335, 261,
                  676, 1493, 46, 1075, 20, 814, 1970, 1873, 2029],
                 dtype=np.int32)


def _tree_sum(xs):
    while len(xs) > 1:
        nxt = [xs[i] + xs[i + 1] for i in range(0, len(xs) - 1, 2)]
        if len(xs) % 2:
            nxt.append(xs[-1])
        xs = nxt
    return xs[0]


def _stage1_body(qt_ref, kt_ref, ohh_ref, ohl_ref, m_ref,
                 *, seq_len, n_heads, n_top, ppb):
    HV = n_heads
    D = qt_ref.shape[2]
    LBLK = qt_ref.shape[3] // ppb
    inv = 1.0 / seq_len
    oh_h = ohh_ref[...]
    oh_l = ohl_ref[...]
    for p in range(ppb):
        kt2 = kt_ref[0][:, :, p * LBLK:(p + 1) * LBLK].reshape(HV * D, LBLK)
        Gk = lax.dot_general(oh_l, kt2, (((1,), (1,)), ((), ())),
                             preferred_element_type=jnp.float32)
        ksb = _tree_sum([oh_h[:, h][:, None] * Gk[:, h * D:(h + 1) * D]
                         for h in range(HV)])
        for h in range(HV):
            qt_h = qt_ref[0, h, :, p * LBLK:(p + 1) * LBLK]
            s = lax.dot_general(ksb, qt_h, (((1,), (0,)), ((), ())),
                                preferred_element_type=jnp.float32)
            m_ref[p, 0, pl.ds(h * LBLK, LBLK)] = (
                jnp.max(s, axis=0) - jnp.sum(s, axis=0) * inv)


def _stage1(qt, kt):
    B, HV, D, L = qt.shape
    P = B * HV
    LBLK = L // HV
    U = 40
    PPB = 8
    NB = HV // PPB
    ohh = jnp.asarray(np.equal(np.arange(HV)[None, :],
                               (_SAMP[:U] % HV)[:, None]).astype(np.float32))
    ohl = jnp.asarray(np.equal(np.arange(LBLK)[None, :],
                               (_SAMP[:U] // HV)[:, None]).astype(np.float32))
    return pl.pallas_call(
        functools.partial(_stage1_body, seq_len=L, n_heads=HV, n_top=U,
                          ppb=PPB),
        grid=(P // PPB,),
        in_specs=[
            pl.BlockSpec((1, HV, D, PPB * LBLK),
                         lambda i: (i // NB, 0, 0, i % NB)),
            pl.BlockSpec((1, HV, D, PPB * LBLK),
                         lambda i: (i // NB, 0, 0, i % NB)),
            pl.BlockSpec((U, HV), lambda i: (0, 0)),
            pl.BlockSpec((U, LBLK), lambda i: (0, 0)),
        ],
        out_specs=pl.BlockSpec((PPB, 1, L), lambda i: (i, 0, 0)),
        out_shape=jax.ShapeDtypeStruct((P, 1, L), jnp.float32),
    )(qt, kt, ohh, ohl)


def _lane0_mask():
    return jnp.arange(16, dtype=jnp.int32) == 0


def _store_scalar(ref, pos, val):
    idx = jnp.full((16,), pos, dtype=jnp.int32)
    x = jnp.full((16,), val, dtype=ref.dtype)
    plsc.store_scatter(ref, [idx], x, mask=_lane0_mask())


def _sc_body(m_hbm, idx_hbm, row_v, cmax_v, idx_v,
             *, seq_len, n_top, lblk, pairs_per_subcore):
    n_chunks = seq_len // 16
    cvecs = n_chunks // 16
    wid = lax.axis_index("s") * 2 + lax.axis_index("c")

    for p in range(pairs_per_subcore):
        pair = wid * pairs_per_subcore + p
        pltpu.sync_copy(m_hbm.at[pair, 0], row_v)

        def _init(j, carry):
            v = row_v[pl.ds(j * 16, 16)]
            _store_scalar(cmax_v, j, jnp.max(v))
            return carry
        lax.fori_loop(0, n_chunks, _init, 0)

        for t in range(n_top // 16, 8):
            idx_v[pl.ds(t * 16, 16)] = jnp.zeros((16,), jnp.int32)

        def _extract(t, carry):
            rm = cmax_v[pl.ds(0, 16)]
            for j in range(1, cvecs):
                rm = jnp.maximum(rm, cmax_v[pl.ds(j * 16, 16)])
            gmax = jnp.max(rm)

            def _find(j, best):
                mj = cmax_v[pl.ds(j * 16, 16)] == gmax
                fj = plsc.all_reduce_ffs(mj)
                fj = jnp.asarray(fj).reshape(-1)[0]
                cand = j * 16 + fj
                hit = (fj < 16) & (best >= n_chunks)
                return jnp.where(hit, cand, best)
            chunk = lax.fori_loop(0, cvecs, _find, jnp.int32(n_chunks))

            v = row_v[pl.ds(chunk * 16, 16)]
            lane = plsc.all_reduce_ffs(v == gmax)
            lane = jnp.asarray(lane).reshape(-1)[0]
            elem = chunk * 16 + lane
            vrow = (elem % lblk) * (seq_len // lblk) + elem // lblk
            _store_scalar(idx_v, t, vrow)
            v2 = jnp.where(jnp.arange(16, dtype=jnp.int32) == lane, _NEG, v)
            row_v[pl.ds(chunk * 16, 16)] = v2
            _store_scalar(cmax_v, chunk, jnp.max(v2))
            return carry
        lax.fori_loop(0, n_top, _extract, 0)

        pltpu.sync_copy(idx_v, idx_hbm.at[pair])


def _sc_topk(m, lblk):
    P, _, L = m.shape
    U = 40
    pps = P // 32
    mesh = plsc.VectorSubcoreMesh(core_axis_name="c", subcore_axis_name="s")
    fn = pl.kernel(
        functools.partial(_sc_body, seq_len=L, n_top=U, lblk=lblk,
                          pairs_per_subcore=pps),
        out_type=jax.ShapeDtypeStruct((P, 128), jnp.int32),
        mesh=mesh,
        compiler_params=pltpu.CompilerParams(needs_layout_passes=False),
        scratch_types=[
            pltpu.VMEM((L,), jnp.float32),
            pltpu.VMEM((L // 16,), jnp.float32),
            pltpu.VMEM((128,), jnp.int32),
        ],
    )
    return fn(m)


def _stage2_body(kt_ref, vt_ref, qt_ref, idx_ref, out_ref,
                 *, scale, n_top, ppb):
    HV = kt_ref.shape[1]
    D = kt_ref.shape[2]
    LBLK = kt_ref.shape[3] // ppb
    L = HV * LBLK
    for p in range(ppb):
        idxv = idx_ref[p, 0][:n_top]
        hh = idxv % HV
        ll = idxv // HV

        oh_h = (lax.broadcasted_iota(jnp.int32, (n_top, HV), 1)
                == hh[:, None]).astype(jnp.float32)
        oh_l = (lax.broadcasted_iota(jnp.int32, (n_top, LBLK), 1)
                == ll[:, None]).astype(jnp.float32)

        qt2 = qt_ref[0][:, :, p * LBLK:(p + 1) * LBLK].reshape(HV * D, LBLK)
        G = lax.dot_general(oh_l, qt2, (((1,), (1,)), ((), ())),
                            preferred_element_type=jnp.float32)
        qr = _tree_sum([oh_h[:, h][:, None] * G[:, h * D:(h + 1) * D]
                        for h in range(HV)])
        qr = (qr * scale).astype(jnp.bfloat16)

        S = jnp.concatenate(
            [lax.dot_general(
                qr,
                kt_ref[0, h, :, p * LBLK:(p + 1) * LBLK].astype(jnp.bfloat16),
                (((1,), (0,)), ((), ())),
                preferred_element_type=jnp.float32)
             for h in range(HV)], axis=1)
        mx = jnp.max(S, axis=1, keepdims=True)
        E = jnp.exp(S - mx)
        den = jnp.sum(E, axis=1, keepdims=True)
        A = (E / den).astype(jnp.bfloat16)
        upd = _tree_sum(
            [lax.dot_general(
                A[:, h * LBLK:(h + 1) * LBLK],
                vt_ref[0, h, :, p * LBLK:(p + 1) * LBLK].astype(jnp.bfloat16),
                (((1,), (1,)), ((), ())),
                preferred_element_type=jnp.float32)
             for h in range(HV)])
        vsum = _tree_sum(
            [jnp.sum(vt_ref[0, h, :, p * LBLK:(p + 1) * LBLK], axis=1)[None, :]
             for h in range(HV)])

        oh2 = (lax.broadcasted_iota(jnp.int32, (n_top, L), 1)
               == idxv[:, None]).astype(jnp.float32)
        outT = lax.dot_general(upd - vsum, oh2, (((0,), (0,)), ((), ())),
                               preferred_element_type=jnp.float32)
        out_ref[0, p] = outT + jnp.broadcast_to(vsum.reshape(D, 1), (D, L))


def _stage2(kt, vt, qt, idx3):
    B, HV, D, L = kt.shape
    P = B * HV
    LBLK = L // HV
    U = 40
    PPB = 8
    NB = HV // PPB
    scale = 1.0 / np.sqrt(64)
    return pl.pallas_call(
        functools.partial(_stage2_body, scale=scale, n_top=U, ppb=PPB),
        grid=(P // PPB,),
        in_specs=[
            pl.BlockSpec((1, HV, D, PPB * LBLK),
                         lambda i: (i // NB, 0, 0, i % NB)),
            pl.BlockSpec((1, HV, D, PPB * LBLK),
                         lambda i: (i // NB, 0, 0, i % NB)),
            pl.BlockSpec((1, HV, D, PPB * LBLK),
                         lambda i: (i // NB, 0, 0, i % NB)),
            pl.BlockSpec((PPB, 1, 128), lambda i: (i, 0, 0)),
        ],
        out_specs=pl.BlockSpec((1, PPB, D, L),
                               lambda i: (i // NB, i % NB, 0, 0)),
        out_shape=jax.ShapeDtypeStruct((B, HV, D, L), jnp.float32),
    )(kt, vt, qt, idx3)


def kernel(queries, keys, values):
    B, L, H, D = queries.shape
    S = keys.shape[1]
    HV = H * D // 64
    LBLK = L // HV
    P = B * HV

    qt = jnp.transpose(queries, (0, 2, 3, 1))
    kt = jnp.transpose(keys, (0, 2, 3, 1))
    vt = jnp.transpose(values, (0, 2, 3, 1))

    m = _stage1(qt, kt)
    idx = _sc_topk(m, LBLK)
    ctx = _stage2(kt, vt, qt, idx.reshape(P, 1, 128))
    return jnp.transpose(ctx, (0, 1, 3, 2))


# speedup vs baseline: 2.8103x; 1.1624x over previous
import functools

import jax
import jax.numpy as jnp
from jax import lax
from jax.experimental import pallas as pl
from jax.experimental.pallas import tpu as pltpu
from jax.experimental.pallas import tpu_sc as plsc

import numpy as np

_FACTOR = 5
_NEG = np.float32(-3.0e38)

_SAMP = np.array([1220, 18, 1207, 1217, 653, 1387, 385, 295, 6, 1282, 552,
                  2034, 1433, 475, 1996, 1810, 1611, 898, 835, 519, 1590,
                  651, 268, 1731, 1132, 1553, 1008, 539, 284, 1335, 261,
                  676, 1493, 46, 1075, 20, 814, 1970, 1873, 2029],
                 dtype=np.int32)


def _tree_sum(xs):
    while len(xs) > 1:
        nxt = [xs[i] + xs[i + 1] for i in range(0, len(xs) - 1, 2)]
        if len(xs) % 2:
            nxt.append(xs[-1])
        xs = nxt
    return xs[0]


def _stage1_body(qt_ref, kt_ref, ohh_ref, ohl_ref, m_ref,
                 *, seq_len, n_heads, n_top, ppb):
    HV = n_heads
    D = qt_ref.shape[2]
    LBLK = qt_ref.shape[3] // ppb
    inv = 1.0 / seq_len
    oh_h = ohh_ref[...]
    oh_l = ohl_ref[...]
    for p in range(ppb):
        kt2 = kt_ref[0][:, :, p * LBLK:(p + 1) * LBLK].reshape(HV * D, LBLK)
        Gk = lax.dot_general(oh_l, kt2, (((1,), (1,)), ((), ())),
                             preferred_element_type=jnp.float32)
        ksb = _tree_sum([oh_h[:, h][:, None] * Gk[:, h * D:(h + 1) * D]
                         for h in range(HV)])
        for h in range(HV):
            qt_h = qt_ref[0, h, :, p * LBLK:(p + 1) * LBLK]
            s = lax.dot_general(ksb, qt_h, (((1,), (0,)), ((), ())),
                                preferred_element_type=jnp.float32)
            m_ref[p, 0, pl.ds(h * LBLK, LBLK)] = (
                jnp.max(s, axis=0) - jnp.sum(s, axis=0) * inv)


def _stage1(qt, kt):
    B, HV, D, L = qt.shape
    P = B * HV
    LBLK = L // HV
    U = 40
    PPB = 8
    NB = HV // PPB
    ohh = jnp.asarray(np.equal(np.arange(HV)[None, :],
                               (_SAMP[:U] % HV)[:, None]).astype(np.float32))
    ohl = jnp.asarray(np.equal(np.arange(LBLK)[None, :],
                               (_SAMP[:U] // HV)[:, None]).astype(np.float32))
    return pl.pallas_call(
        functools.partial(_stage1_body, seq_len=L, n_heads=HV, n_top=U,
                          ppb=PPB),
        grid=(P // PPB,),
        in_specs=[
            pl.BlockSpec((1, HV, D, PPB * LBLK),
                         lambda i: (i // NB, 0, 0, i % NB)),
            pl.BlockSpec((1, HV, D, PPB * LBLK),
                         lambda i: (i // NB, 0, 0, i % NB)),
            pl.BlockSpec((U, HV), lambda i: (0, 0)),
            pl.BlockSpec((U, LBLK), lambda i: (0, 0)),
        ],
        out_specs=pl.BlockSpec((PPB, 1, L), lambda i: (i, 0, 0)),
        out_shape=jax.ShapeDtypeStruct((P, 1, L), jnp.float32),
    )(qt, kt, ohh, ohl)


def _lane0_mask():
    return jnp.arange(16, dtype=jnp.int32) == 0


def _store_scalar(ref, pos, val):
    idx = jnp.full((16,), pos, dtype=jnp.int32)
    x = jnp.full((16,), val, dtype=ref.dtype)
    plsc.store_scatter(ref, [idx], x, mask=_lane0_mask())


def _sc_body(m_hbm, idx_hbm, row_v, cmax_v, idx_v,
             *, seq_len, n_top, lblk, pairs_per_subcore):
    n_chunks = seq_len // 16
    cvecs = n_chunks // 16
    wid = lax.axis_index("s") * 2 + lax.axis_index("c")

    for p in range(pairs_per_subcore):
        pair = wid * pairs_per_subcore + p
        pltpu.sync_copy(m_hbm.at[pair, 0], row_v)

        def _init(j, carry):
            v = row_v[pl.ds(j * 16, 16)]
            _store_scalar(cmax_v, j, jnp.max(v))
            return carry
        lax.fori_loop(0, n_chunks, _init, 0)

        for t in range(n_top // 16, 8):
            idx_v[pl.ds(t * 16, 16)] = jnp.zeros((16,), jnp.int32)

        def _extract(t, carry):
            rm = cmax_v[pl.ds(0, 16)]
            for j in range(1, cvecs):
                rm = jnp.maximum(rm, cmax_v[pl.ds(j * 16, 16)])
            gmax = jnp.max(rm)

            def _find(j, best):
                mj = cmax_v[pl.ds(j * 16, 16)] == gmax
                fj = plsc.all_reduce_ffs(mj)
                fj = jnp.asarray(fj).reshape(-1)[0]
                cand = j * 16 + fj
                hit = (fj < 16) & (best >= n_chunks)
                return jnp.where(hit, cand, best)
            chunk = lax.fori_loop(0, cvecs, _find, jnp.int32(n_chunks))

            v = row_v[pl.ds(chunk * 16, 16)]
            lane = plsc.all_reduce_ffs(v == gmax)
            lane = jnp.asarray(lane).reshape(-1)[0]
            elem = chunk * 16 + lane
            vrow = (elem % lblk) * (seq_len // lblk) + elem // lblk
            _store_scalar(idx_v, t, vrow)
            v2 = jnp.where(jnp.arange(16, dtype=jnp.int32) == lane, _NEG, v)
            row_v[pl.ds(chunk * 16, 16)] = v2
            _store_scalar(cmax_v, chunk, jnp.max(v2))
            return carry
        lax.fori_loop(0, n_top, _extract, 0)

        pltpu.sync_copy(idx_v, idx_hbm.at[pair])


def _sc_topk(m, lblk):
    P, _, L = m.shape
    U = 40
    pps = P // 32
    mesh = plsc.VectorSubcoreMesh(core_axis_name="c", subcore_axis_name="s")
    fn = pl.kernel(
        functools.partial(_sc_body, seq_len=L, n_top=U, lblk=lblk,
                          pairs_per_subcore=pps),
        out_type=jax.ShapeDtypeStruct((P, 128), jnp.int32),
        mesh=mesh,
        compiler_params=pltpu.CompilerParams(needs_layout_passes=False),
        scratch_types=[
            pltpu.VMEM((L,), jnp.float32),
            pltpu.VMEM((L // 16,), jnp.float32),
            pltpu.VMEM((128,), jnp.int32),
        ],
    )
    return fn(m)


def _stage2_body(kt_ref, vt_ref, qt_ref, idx_ref, out_ref,
                 *, scale, n_top, ppb):
    HV = kt_ref.shape[1]
    D = kt_ref.shape[2]
    LBLK = kt_ref.shape[3] // ppb
    L = HV * LBLK
    S_all, idx_all = [], []
    for p in range(ppb):
        idxv = idx_ref[p, 0][:n_top]
        hh = idxv % HV
        ll = idxv // HV

        oh_h = (lax.broadcasted_iota(jnp.int32, (n_top, HV), 1)
                == hh[:, None]).astype(jnp.float32)
        oh_l = (lax.broadcasted_iota(jnp.int32, (n_top, LBLK), 1)
                == ll[:, None]).astype(jnp.float32)

        qt2 = qt_ref[0][:, :, p * LBLK:(p + 1) * LBLK].reshape(HV * D, LBLK)
        G = lax.dot_general(oh_l, qt2, (((1,), (1,)), ((), ())),
                            preferred_element_type=jnp.float32)
        qr = _tree_sum([oh_h[:, h][:, None] * G[:, h * D:(h + 1) * D]
                        for h in range(HV)])
        qr = (qr * scale).astype(jnp.bfloat16)

        S = jnp.concatenate(
            [lax.dot_general(
                qr,
                kt_ref[0, h, :, p * LBLK:(p + 1) * LBLK].astype(jnp.bfloat16),
                (((1,), (0,)), ((), ())),
                preferred_element_type=jnp.float32)
             for h in range(HV)], axis=1)
        S_all.append(S)
        idx_all.append(idxv)

    for p in range(ppb):
        S, idxv = S_all[p], idx_all[p]
        mx = jnp.max(S, axis=1, keepdims=True)
        E = jnp.exp(S - mx)
        den = jnp.sum(E, axis=1, keepdims=True)
        A = (E / den).astype(jnp.bfloat16)
        upd = _tree_sum(
            [lax.dot_general(
                A[:, h * LBLK:(h + 1) * LBLK],
                vt_ref[0, h, :, p * LBLK:(p + 1) * LBLK].astype(jnp.bfloat16),
                (((1,), (1,)), ((), ())),
                preferred_element_type=jnp.float32)
             for h in range(HV)])
        vsum = _tree_sum(
            [jnp.sum(vt_ref[0, h, :, p * LBLK:(p + 1) * LBLK], axis=1)[None, :]
             for h in range(HV)])

        oh2 = (lax.broadcasted_iota(jnp.int32, (n_top, L), 1)
               == idxv[:, None]).astype(jnp.float32)
        outT = lax.dot_general(upd - vsum, oh2, (((0,), (0,)), ((), ())),
                               preferred_element_type=jnp.float32)
        out_ref[0, p] = outT + jnp.broadcast_to(vsum.reshape(D, 1), (D, L))


def _stage2(kt, vt, qt, idx3):
    B, HV, D, L = kt.shape
    P = B * HV
    LBLK = L // HV
    U = 40
    PPB = 8
    NB = HV // PPB
    scale = 1.0 / np.sqrt(64)
    return pl.pallas_call(
        functools.partial(_stage2_body, scale=scale, n_top=U, ppb=PPB),
        grid=(P // PPB,),
        in_specs=[
            pl.BlockSpec((1, HV, D, PPB * LBLK),
                         lambda i: (i // NB, 0, 0, i % NB)),
            pl.BlockSpec((1, HV, D, PPB * LBLK),
                         lambda i: (i // NB, 0, 0, i % NB)),
            pl.BlockSpec((1, HV, D, PPB * LBLK),
                         lambda i: (i // NB, 0, 0, i % NB)),
            pl.BlockSpec((PPB, 1, 128), lambda i: (i, 0, 0)),
        ],
        out_specs=pl.BlockSpec((1, PPB, D, L),
                               lambda i: (i // NB, i % NB, 0, 0)),
        out_shape=jax.ShapeDtypeStruct((B, HV, D, L), jnp.float32),
    )(kt, vt, qt, idx3)


def kernel(queries, keys, values):
    B, L, H, D = queries.shape
    S = keys.shape[1]
    HV = H * D // 64
    LBLK = L // HV
    P = B * HV

    qt = jnp.transpose(queries, (0, 2, 3, 1))
    kt = jnp.transpose(keys, (0, 2, 3, 1))
    vt = jnp.transpose(values, (0, 2, 3, 1))

    m = _stage1(qt, kt)
    idx = _sc_topk(m, LBLK)
    ctx = _stage2(kt, vt, qt, idx.reshape(P, 1, 128))
    return jnp.transpose(ctx, (0, 1, 3, 2))
